# baseline (device time: 56370 ns/iter reference)
import jax
import jax.numpy as jnp
from jax import lax
from jax.experimental import pallas as pl
from jax.experimental.pallas import tpu as pltpu

N_DEV = 16
SQ = 256
D = 1024
DH = 128
NH_LOCAL = 8
SCALE = 0.08838834764831843


def kernel(x, Wq, Wo, Wk, Wv):
    def body(
        x_ref,
        wq_ref,
        wo_ref,
        wk_ref,
        wv_ref,
        out_ref,
        part_ref,
        buf_x,
        buf_y,
        buf_z0,
        buf_z1,
        rs_send_sems,
        rs_recv_sems,
        ag_send_sems,
        ag_recv_sems,
    ):
        p = lax.axis_index("i")
        z = p // 4
        i = lax.rem(p, 4)
        xb = jnp.where((i == 1) | (i == 2), 1, 0).astype(jnp.int32)
        yb = i // 2
        zl = lax.rem(z, 2)
        zh = z // 2

        p_x = p + 1 - 2 * lax.rem(i, 2)
        p_y = p + 3 - 2 * i
        p_z0 = p + 4 * (1 - 2 * zl)
        p_z1 = p + 8 * (1 - 2 * zh)

        bf16 = jnp.bfloat16
        xm = x_ref[0].astype(bf16)
        q = jnp.dot(xm, wq_ref[...].astype(bf16), preferred_element_type=jnp.float32)
        k = jnp.dot(xm, wk_ref[...].astype(bf16), preferred_element_type=jnp.float32)
        v = jnp.dot(xm, wv_ref[...].astype(bf16), preferred_element_type=jnp.float32)
        q16 = q.astype(bf16)
        k16 = k.astype(bf16)
        v16 = v.astype(bf16)
        wo16 = wo_ref[...].astype(bf16)

        part = jnp.zeros((SQ, D), jnp.float32)
        for h in range(NH_LOCAL):
            sl = slice(h * DH, (h + 1) * DH)
            s = (
                lax.dot_general(
                    q16[:, sl],
                    k16[:, sl],
                    (((1,), (1,)), ((), ())),
                    preferred_element_type=jnp.float32,
                )
                * SCALE
            )
            m = jnp.max(s, axis=1, keepdims=True)
            pr = jnp.exp(s - m)
            l = jnp.sum(pr, axis=1, keepdims=True)
            o = jnp.dot(
                pr.astype(bf16), v16[:, sl], preferred_element_type=jnp.float32
            ) / l
            part = part + jnp.dot(
                o.astype(bf16), wo16[sl, :], preferred_element_type=jnp.float32
            )
        part_ref[...] = part

        barrier = pltpu.get_barrier_semaphore()
        for nbr in (p_x, p_y, p_z0, p_z1):
            pl.semaphore_signal(
                barrier,
                inc=1,
                device_id=(nbr,),
                device_id_type=pl.DeviceIdType.MESH,
            )
        pl.semaphore_wait(barrier, 4)

        rs_steps = [
            (128, p_x, xb, buf_x),
            (64, p_y, yb, buf_y),
            (32, p_z0, zl, buf_z0),
            (16, p_z1, zh, buf_z1),
        ]
        base = jnp.int32(0)
        for step, (h, partner, vb, buf) in enumerate(rs_steps):
            send_off = base + (1 - vb) * h
            keep_off = base + vb * h
            rdma = pltpu.make_async_remote_copy(
                src_ref=part_ref.at[pl.ds(send_off, h), :],
                dst_ref=buf.at[:, :],
                send_sem=rs_send_sems.at[step],
                recv_sem=rs_recv_sems.at[step],
                device_id=(partner,),
                device_id_type=pl.DeviceIdType.MESH,
            )
            rdma.start()
            rdma.wait()
            rows = pl.ds(keep_off, h)
            part_ref[rows, :] = part_ref[rows, :] + buf[:, :]
            base = keep_off

        out_ref[0, pl.ds(base, 16), :] = part_ref[pl.ds(base, 16), :]

        ag_steps = [
            (16, p_z1, zh),
            (32, p_z0, zl),
            (64, p_y, yb),
            (128, p_x, xb),
        ]
        for step, (h, partner, vb) in enumerate(ag_steps):
            pbase = base + (1 - 2 * vb) * h
            rdma = pltpu.make_async_remote_copy(
                src_ref=out_ref.at[0, pl.ds(base, h), :],
                dst_ref=out_ref.at[0, pl.ds(base, h), :],
                send_sem=ag_send_sems.at[step],
                recv_sem=ag_recv_sems.at[step],
                device_id=(partner,),
                device_id_type=pl.DeviceIdType.MESH,
            )
            rdma.start()
            rdma.wait()
            base = base - vb * h

    return pl.pallas_call(
        body,
        out_shape=jax.ShapeDtypeStruct((1, SQ, D), jnp.float32),
        in_specs=[pl.BlockSpec(memory_space=pltpu.VMEM)] * 5,
        out_specs=pl.BlockSpec(memory_space=pltpu.VMEM),
        scratch_shapes=[
            pltpu.VMEM((SQ, D), jnp.float32),
            pltpu.VMEM((128, D), jnp.float32),
            pltpu.VMEM((64, D), jnp.float32),
            pltpu.VMEM((32, D), jnp.float32),
            pltpu.VMEM((16, D), jnp.float32),
            pltpu.SemaphoreType.DMA((4,)),
            pltpu.SemaphoreType.DMA((4,)),
            pltpu.SemaphoreType.DMA((4,)),
            pltpu.SemaphoreType.DMA((4,)),
        ],
        compiler_params=pltpu.CompilerParams(collective_id=0),
    )(x, Wq, Wo, Wk, Wv)


# device time: 36465 ns/iter; 1.5459x vs baseline; 1.5459x over previous
import jax
import jax.numpy as jnp
from jax import lax
from jax.experimental import pallas as pl
from jax.experimental.pallas import tpu as pltpu

N_DEV = 16
SQ = 256
D = 1024
DH = 128
NH_LOCAL = 8
CH = SQ // N_DEV
SCALE = 0.08838834764831843


def kernel(x, Wq, Wo, Wk, Wv):
    def body(
        x_ref,
        wq_ref,
        wo_ref,
        wk_ref,
        wv_ref,
        out_ref,
        part16_ref,
        rs_buf,
        ag_buf,
        rs_send_sems,
        rs_recv_sems,
        ag_send_sems,
        ag_recv_sems,
    ):
        p = lax.axis_index("i")
        bf16 = jnp.bfloat16

        xm = x_ref[0].astype(bf16)
        q = jnp.dot(xm, wq_ref[...].astype(bf16), preferred_element_type=jnp.float32)
        k = jnp.dot(xm, wk_ref[...].astype(bf16), preferred_element_type=jnp.float32)
        v = jnp.dot(xm, wv_ref[...].astype(bf16), preferred_element_type=jnp.float32)
        q16 = q.astype(bf16)
        k16 = k.astype(bf16)
        v16 = v.astype(bf16)
        wo16 = wo_ref[...].astype(bf16)

        part = jnp.zeros((SQ, D), jnp.float32)
        for h in range(NH_LOCAL):
            sl = slice(h * DH, (h + 1) * DH)
            s = (
                lax.dot_general(
                    q16[:, sl],
                    k16[:, sl],
                    (((1,), (1,)), ((), ())),
                    preferred_element_type=jnp.float32,
                )
                * SCALE
            )
            m = jnp.max(s, axis=1, keepdims=True)
            pr = jnp.exp(s - m)
            l = jnp.sum(pr, axis=1, keepdims=True)
            o = jnp.dot(
                pr.astype(bf16), v16[:, sl], preferred_element_type=jnp.float32
            ) / l
            part = part + jnp.dot(
                o.astype(bf16), wo16[sl, :], preferred_element_type=jnp.float32
            )
        part16_ref[...] = part.astype(bf16)
        rs_buf[p, :, :] = part16_ref[pl.ds(p * CH, CH), :]

        barrier = pltpu.get_barrier_semaphore()
        for j in range(N_DEV - 1):
            pl.semaphore_signal(
                barrier,
                inc=1,
                device_id=(lax.rem(p + 1 + j, N_DEV),),
                device_id_type=pl.DeviceIdType.MESH,
            )
        pl.semaphore_wait(barrier, N_DEV - 1)

        rs_rdmas = []
        for j in range(N_DEV - 1):
            tgt = lax.rem(p + 1 + j, N_DEV)
            rdma = pltpu.make_async_remote_copy(
                src_ref=part16_ref.at[pl.ds(tgt * CH, CH), :],
                dst_ref=rs_buf.at[p],
                send_sem=rs_send_sems.at[j],
                recv_sem=rs_recv_sems.at[j],
                device_id=(tgt,),
                device_id_type=pl.DeviceIdType.MESH,
            )
            rdma.start()
            rs_rdmas.append(rdma)
        for rdma in rs_rdmas:
            rdma.wait_recv()

        red = rs_buf[0].astype(jnp.float32)
        for s_ in range(1, N_DEV):
            red = red + rs_buf[s_].astype(jnp.float32)
        myrows = pl.ds(p * CH, CH)
        ag_buf[myrows, :] = red.astype(bf16)

        ag_rdmas = []
        for j in range(N_DEV - 1):
            tgt = lax.rem(p + 1 + j, N_DEV)
            rdma = pltpu.make_async_remote_copy(
                src_ref=ag_buf.at[myrows, :],
                dst_ref=ag_buf.at[myrows, :],
                send_sem=ag_send_sems.at[j],
                recv_sem=ag_recv_sems.at[j],
                device_id=(tgt,),
                device_id_type=pl.DeviceIdType.MESH,
            )
            rdma.start()
            ag_rdmas.append(rdma)
        for rdma in ag_rdmas:
            rdma.wait_recv()

        out_ref[0] = ag_buf[...].astype(jnp.float32)
        out_ref[0, myrows, :] = red

        for rdma in rs_rdmas:
            rdma.wait_send()
        for rdma in ag_rdmas:
            rdma.wait_send()

    return pl.pallas_call(
        body,
        out_shape=jax.ShapeDtypeStruct((1, SQ, D), jnp.float32),
        in_specs=[pl.BlockSpec(memory_space=pltpu.VMEM)] * 5,
        out_specs=pl.BlockSpec(memory_space=pltpu.VMEM),
        scratch_shapes=[
            pltpu.VMEM((SQ, D), jnp.bfloat16),
            pltpu.VMEM((N_DEV, CH, D), jnp.bfloat16),
            pltpu.VMEM((SQ, D), jnp.bfloat16),
            pltpu.SemaphoreType.DMA((N_DEV - 1,)),
            pltpu.SemaphoreType.DMA((N_DEV - 1,)),
            pltpu.SemaphoreType.DMA((N_DEV - 1,)),
            pltpu.SemaphoreType.DMA((N_DEV - 1,)),
        ],
        compiler_params=pltpu.CompilerParams(collective_id=0),
    )(x, Wq, Wo, Wk, Wv)
